# Initial kernel scaffold; baseline (speedup 1.0000x reference)
import functools

import jax
import jax.numpy as jnp
from jax import lax
from jax.experimental import pallas as pl
from jax.experimental.pallas import tpu as pltpu
from jax.experimental.pallas import tpu_sc as plsc

_BATCH = 16384
_NVARS = 3
_STRIDE0 = 10000
_STRIDE1 = 100
_NC = 2
_NS = 16
_NW = _NC * _NS
_BPW = _BATCH // _NW
_GCHUNK = 128
_NGATHER = _BPW // _GCHUNK


def _sc_lookup_body(xs_hbm, table_hbm, out_hbm, xs_v, idx_v, out_v, sem):
    wid = lax.axis_index("s") * _NC + lax.axis_index("c")
    base = wid * _BPW

    pltpu.sync_copy(xs_hbm.at[pl.ds(base * _NVARS, _BPW * _NVARS)], xs_v)

    lane3 = lax.iota(jnp.int32, 16) * _NVARS
    for c in range(_BPW // 16):
        g = lane3 + (c * 16 * _NVARS)
        x0 = plsc.load_gather(xs_v, [g])
        x1 = plsc.load_gather(xs_v, [g + 1])
        x2 = plsc.load_gather(xs_v, [g + 2])
        idx = x0 * _STRIDE0 + x1 * _STRIDE1 + x2
        idx_v[c // 8, pl.ds((c % 8) * 16, 16)] = idx

    copies = [
        pltpu.async_copy(table_hbm.at[idx_v.at[k]], out_v.at[k], sem)
        for k in range(_NGATHER)
    ]
    for cp in copies:
        cp.wait()

    for k in range(_NGATHER):
        pltpu.sync_copy(out_v.at[k], out_hbm.at[pl.ds(base + k * _GCHUNK, _GCHUNK)])


@functools.partial(
    pl.kernel,
    out_type=jax.ShapeDtypeStruct((_BATCH,), jnp.float32),
    mesh=plsc.VectorSubcoreMesh(
        core_axis_name="c", subcore_axis_name="s", num_cores=_NC, num_subcores=_NS
    ),
    scratch_types=[
        pltpu.VMEM((_BPW * _NVARS,), jnp.int32),
        pltpu.VMEM((_NGATHER, _GCHUNK), jnp.int32),
        pltpu.VMEM((_NGATHER, _GCHUNK), jnp.float32),
        pltpu.SemaphoreType.DMA,
    ],
)
def _sc_lookup(xs_hbm, table_hbm, out_hbm, xs_v, idx_v, out_v, sem):
    _sc_lookup_body(xs_hbm, table_hbm, out_hbm, xs_v, idx_v, out_v, sem)


def kernel(xs, embed_weight):
    xs_flat = xs.reshape(-1)
    table = embed_weight.reshape(-1)
    return _sc_lookup(xs_flat, table)


# speedup vs baseline: 1.0122x; 1.0122x over previous
import functools

import jax
import jax.numpy as jnp
from jax import lax
from jax.experimental import pallas as pl
from jax.experimental.pallas import tpu as pltpu
from jax.experimental.pallas import tpu_sc as plsc

_BATCH = 16384
_STRIDE0 = 10000
_STRIDE1 = 100
_NC = 2
_NS = 16
_NW = _NC * _NS
_BPW = _BATCH // _NW
_GCHUNK = 128
_NGATHER = _BPW // _GCHUNK


def _sc_lookup_body(xs_hbm, table_hbm, out_hbm, x0_v, x1_v, x2_v, idx_v, out_v, sem):
    wid = lax.axis_index("s") * _NC + lax.axis_index("c")
    base = wid * _BPW

    pltpu.sync_copy(xs_hbm.at[pl.ds(base, _BPW)], x0_v)
    pltpu.sync_copy(xs_hbm.at[pl.ds(_BATCH + base, _BPW)], x1_v)
    pltpu.sync_copy(xs_hbm.at[pl.ds(2 * _BATCH + base, _BPW)], x2_v)

    for c in range(_BPW // 16):
        s = pl.ds(c * 16, 16)
        idx = x0_v[s] * _STRIDE0 + x1_v[s] * _STRIDE1 + x2_v[s]
        idx_v[c // 8, pl.ds((c % 8) * 16, 16)] = idx

    copies = [
        pltpu.async_copy(table_hbm.at[idx_v.at[k]], out_v.at[k], sem)
        for k in range(_NGATHER)
    ]
    for cp in copies:
        cp.wait()

    for k in range(_NGATHER):
        pltpu.sync_copy(out_v.at[k], out_hbm.at[pl.ds(base + k * _GCHUNK, _GCHUNK)])


@functools.partial(
    pl.kernel,
    out_type=jax.ShapeDtypeStruct((_BATCH,), jnp.float32),
    mesh=plsc.VectorSubcoreMesh(
        core_axis_name="c", subcore_axis_name="s", num_cores=_NC, num_subcores=_NS
    ),
    scratch_types=[
        pltpu.VMEM((_BPW,), jnp.int32),
        pltpu.VMEM((_BPW,), jnp.int32),
        pltpu.VMEM((_BPW,), jnp.int32),
        pltpu.VMEM((_NGATHER, _GCHUNK), jnp.int32),
        pltpu.VMEM((_NGATHER, _GCHUNK), jnp.float32),
        pltpu.SemaphoreType.DMA,
    ],
)
def _sc_lookup(xs_hbm, table_hbm, out_hbm, x0_v, x1_v, x2_v, idx_v, out_v, sem):
    _sc_lookup_body(xs_hbm, table_hbm, out_hbm, x0_v, x1_v, x2_v, idx_v, out_v, sem)


def kernel(xs, embed_weight):
    xs_t = xs.T.reshape(-1)
    table = embed_weight.reshape(-1)
    return _sc_lookup(xs_t, table)
